# baseline (speedup 1.0000x reference)
import functools

import jax
import jax.numpy as jnp
from jax import lax
from jax.experimental import pallas as pl
from jax.experimental.pallas import tpu as pltpu
from jax.experimental.pallas import tpu_sc as plsc

_N = 10000
_E = 160000
_ROWS = 10240
_NSUB = 16
_EPT = _E // _NSUB
_B = 16


def _lay1_body(x_ref, w_ref, bs_ref, bd_ref, ext_ref, adst_ref):
    t = jnp.dot(x_ref[...], w_ref[...], preferred_element_type=jnp.float32)
    asrc = jnp.dot(t, bs_ref[...], preferred_element_type=jnp.float32)
    ext_ref[...] = jnp.concatenate((t, asrc), axis=1)
    adst_ref[...] = jnp.dot(t, bd_ref[...], preferred_element_type=jnp.float32)


def _lay1(x, w, bsrc, bdst, bm=256):
    m, k = x.shape
    n = w.shape[1]
    return pl.pallas_call(
        _lay1_body,
        grid=(m // bm,),
        in_specs=[pl.BlockSpec((bm, k), lambda i: (i, 0)),
                  pl.BlockSpec((k, n), lambda i: (0, 0)),
                  pl.BlockSpec((n, 128), lambda i: (0, 0)),
                  pl.BlockSpec((n, 128), lambda i: (0, 0))],
        out_specs=[pl.BlockSpec((bm, n + 128), lambda i: (i, 0)),
                   pl.BlockSpec((bm, 128), lambda i: (i, 0))],
        out_shape=[jax.ShapeDtypeStruct((m, n + 128), jnp.float32),
                   jax.ShapeDtypeStruct((m, 128), jnp.float32)],
    )(x, w, bsrc, bdst)


def _lay2_body(acc_ref, s_ref, b_ref, w_ref, bs_ref, bd_ref,
               ext_ref, adst_ref):
    a = acc_ref[...]
    dr = jnp.dot(a[:, 1024:1040], s_ref[...],
                 preferred_element_type=jnp.float32) + 1e-16
    t = a[:, :1024] / dr + b_ref[...]
    t = jnp.where(t > 0, t, jnp.exp(jnp.minimum(t, 0.0)) - 1.0)
    h = jnp.dot(t, w_ref[...], preferred_element_type=jnp.float32)
    asrc = jnp.dot(h, bs_ref[...], preferred_element_type=jnp.float32)
    ext_ref[...] = jnp.concatenate((h, asrc), axis=1)
    adst_ref[...] = jnp.dot(h, bd_ref[...], preferred_element_type=jnp.float32)


def _lay2(acc, smat, bias, w, bsrc, bdst, bm=256):
    m = acc.shape[0]
    n = w.shape[1]
    return pl.pallas_call(
        _lay2_body,
        grid=(m // bm,),
        in_specs=[pl.BlockSpec((bm, 1152), lambda i: (i, 0)),
                  pl.BlockSpec((16, 1024), lambda i: (0, 0)),
                  pl.BlockSpec((1, 1024), lambda i: (0, 0)),
                  pl.BlockSpec((1024, n), lambda i: (0, 0)),
                  pl.BlockSpec((n, 128), lambda i: (0, 0)),
                  pl.BlockSpec((n, 128), lambda i: (0, 0))],
        out_specs=[pl.BlockSpec((bm, n + 128), lambda i: (i, 0)),
                   pl.BlockSpec((bm, 128), lambda i: (i, 0))],
        out_shape=[jax.ShapeDtypeStruct((m, n + 128), jnp.float32),
                   jax.ShapeDtypeStruct((m, 128), jnp.float32)],
    )(acc, smat, bias, w, bsrc, bdst)


def _fin_body(acc_ref, b_ref, o_ref):
    a = acc_ref[...]
    dr = a[:, 256:257] + 1e-16
    o_ref[...] = a[:, :256] / dr + b_ref[...]


def _fin(acc, bias, bm=256):
    m = acc.shape[0]
    return pl.pallas_call(
        _fin_body,
        grid=(m // bm,),
        in_specs=[pl.BlockSpec((bm, 384), lambda i: (i, 0)),
                  pl.BlockSpec((1, 256), lambda i: (0, 0))],
        out_specs=pl.BlockSpec((bm, 256), lambda i: (i, 0)),
        out_shape=jax.ShapeDtypeStruct((m, 256), jnp.float32),
    )(acc, bias)


def _make_sc_pass(F, H, CH, NCH):
    fph = F // H
    NS = F // 128 + 1
    mesh = plsc.VectorSubcoreMesh(core_axis_name="c", subcore_axis_name="s")
    wb = CH // _NSUB

    @functools.partial(
        pl.kernel, mesh=mesh,
        compiler_params=pltpu.CompilerParams(needs_layout_passes=False),
        out_type=tuple(jax.ShapeDtypeStruct((_ROWS, 128), jnp.float32)
                       for _ in range(NS)),
        scratch_types=[
            pltpu.VMEM((_EPT,), jnp.int32),
            pltpu.VMEM((_EPT,), jnp.int32),
            pltpu.VMEM((_EPT + 2 * _B,), jnp.int32),
            pltpu.VMEM((_B,), jnp.int32),
            pltpu.VMEM((_B,), jnp.int32),
            pltpu.VMEM((_B,), jnp.int32),
            pltpu.VMEM((_B, 128), jnp.float32),
        ] + [pltpu.VMEM((_B, 128), jnp.float32) for _ in range(NS)]
          + [pltpu.VMEM_SHARED((CH, 128), jnp.float32) for _ in range(NS)]
          + [
            pltpu.SemaphoreType.DMA,
            pltpu.SemaphoreType.DMA,
            pltpu.SemaphoreType.DMA,
        ])
    def sc_pass(src_hbm, dst_hbm, adst_hbm, z_hbm, *rest):
        ext_s = rest[:NS]
        acc_out = rest[NS:2 * NS]
        (src_vm, dst_vm, sel_vm, srci_vm, dsti_vm, dloc_vm,
         adst_vm) = rest[2 * NS:2 * NS + 7]
        rows_s = rest[2 * NS + 7:3 * NS + 7]
        acc_s = rest[3 * NS + 7:4 * NS + 7]
        sem0, sem1, sem2 = rest[4 * NS + 7:]
        cid = lax.axis_index("c")
        sid = lax.axis_index("s")
        ebase = sid * _EPT
        pltpu.sync_copy(src_hbm.at[pl.ds(ebase, _EPT)], src_vm)
        pltpu.sync_copy(dst_hbm.at[pl.ds(ebase, _EPT)], dst_vm)
        iota16 = lax.iota(jnp.int32, 16)
        r0 = sid * wb

        def chunk_body(k, _):
            c0 = (cid * NCH + k) * CH
            for c in range(NS):
                pltpu.sync_copy(z_hbm.at[pl.ds(r0, wb)],
                                acc_s[c].at[pl.ds(r0, wb)])
            plsc.subcore_barrier()

            def fbody(i, cnt):
                d = dst_vm[pl.ds(i * 16, 16)]
                m = (d >= c0) & (d < c0 + CH)
                mi = m.astype(jnp.int32)
                pos = cnt + plsc.cumsum(mi) - 1
                plsc.store_scatter(sel_vm, [pos], iota16 + i * 16, mask=m)
                return cnt + jnp.sum(mi)
            cnt = lax.fori_loop(0, _EPT // 16, fbody, jnp.int32(0))
            for t in range(_B // 16):
                plsc.store_scatter(sel_vm, [cnt + iota16 + t * 16],
                                   jnp.zeros((16,), jnp.int32))
            nb = (cnt + _B - 1) // _B

            def batch_body(j, _):
                base = j * _B
                for g in range(_B // 16):
                    lidx = sel_vm[pl.ds(base + g * 16, 16)]
                    s = plsc.load_gather(src_vm, [lidx])
                    d = plsc.load_gather(dst_vm, [lidx])
                    dl = jnp.minimum(jnp.maximum(d - c0, 0), CH - 1)
                    srci_vm[pl.ds(g * 16, 16)] = s
                    dsti_vm[pl.ds(g * 16, 16)] = d
                    dloc_vm[pl.ds(g * 16, 16)] = dl
                ca = pltpu.async_copy(adst_hbm.at[dsti_vm], adst_vm, sem0)
                gs = [pltpu.async_copy(ext_s[c].at[srci_vm], rows_s[c], sem1)
                      for c in range(NS)]
                ca.wait()
                for g in gs:
                    g.wait()
                wbuf = rows_s[NS - 1]
                for g in range(_B // 16):
                    rowi = iota16 + g * 16
                    valid = (base + rowi) < cnt
                    for hh in range(H):
                        a_s = plsc.load_gather(
                            wbuf, [rowi, jnp.full((16,), hh, jnp.int32)])
                        a_d = plsc.load_gather(
                            adst_vm, [rowi, jnp.full((16,), hh, jnp.int32)])
                        sc = a_s + a_d
                        sc = jnp.where(sc >= 0, sc, 0.2 * sc)
                        wv = jnp.where(valid, jnp.exp(sc), 0.0)
                        plsc.store_scatter(
                            wbuf, [rowi, jnp.full((16,), hh, jnp.int32)], wv)

                @plsc.parallel_loop(0, _B, 1, unroll=4)
                def _scale(b):
                    wrow = wbuf[b, pl.ds(0, 16)]
                    for c in range(NS - 1):
                        wsv = jnp.full((16,), wrow[(c * 128) // fph],
                                       jnp.float32)
                        for jv in range(8):
                            rows_s[c][b, pl.ds(jv * 16, 16)] = (
                                rows_s[c][b, pl.ds(jv * 16, 16)] * wsv)
                ss = [pltpu.async_copy(rows_s[c], acc_s[c].at[dloc_vm], sem2,
                                       add=True)
                      for c in range(NS)]
                for g in ss:
                    g.wait()
                return 0
            lax.fori_loop(0, nb, batch_body, 0)
            plsc.subcore_barrier()

            for c in range(NS):
                pltpu.sync_copy(acc_s[c].at[pl.ds(r0, wb)],
                                acc_out[c].at[pl.ds(c0 + r0, wb)])
            return 0
        lax.fori_loop(0, NCH, chunk_body, 0)

    return sc_pass


_sc_pass_1 = _make_sc_pass(1024, 8, 128, 40)
_sc_pass_2 = _make_sc_pass(256, 1, 512, 10)


def _logit_mats(a_src, a_dst, f_tot, heads):
    fph = f_tot // heads
    bs = jnp.zeros((f_tot, 128), jnp.float32)
    bd = jnp.zeros((f_tot, 128), jnp.float32)
    for h in range(heads):
        bs = bs.at[h * fph:(h + 1) * fph, h].set(a_src[h])
        bd = bd.at[h * fph:(h + 1) * fph, h].set(a_dst[h])
    return bs, bd


def _spread_mat(f_tot, heads):
    fph = f_tot // heads
    s = jnp.zeros((16, f_tot), jnp.float32)
    for h in range(heads):
        s = s.at[h, h * fph:(h + 1) * fph].set(1.0)
    return s


def kernel(x, adj, W1, a_src1, a_dst1, b1, W2, a_src2, a_dst2, b2):
    src = adj[0]
    dst = adj[1]
    bs1, bd1 = _logit_mats(a_src1, a_dst1, 1024, 8)
    bs2, bd2 = _logit_mats(a_src2, a_dst2, 256, 1)
    s1m = _spread_mat(1024, 8)
    z1 = jnp.zeros((128, 128), jnp.float32)
    z2 = jnp.zeros((512, 128), jnp.float32)

    xp = jnp.pad(x, ((0, 10240 - _N), (0, 0)))
    ext1, ad1 = _lay1(xp, W1, bs1, bd1)
    e1s = [ext1[:, c * 128:(c + 1) * 128] for c in range(9)]
    acc1 = jnp.concatenate(
        _sc_pass_1(src, dst, ad1, z1, *e1s), axis=1)
    ext2, ad2 = _lay2(acc1, s1m, b1.reshape(1, -1), W2, bs2, bd2)
    e2s = [ext2[:, c * 128:(c + 1) * 128] for c in range(3)]
    acc2 = jnp.concatenate(
        _sc_pass_2(src, dst, ad2, z2, *e2s), axis=1)
    out = _fin(acc2, b2.reshape(1, -1))
    return out[:_N]


# speedup vs baseline: 1.0221x; 1.0221x over previous
import functools

import jax
import jax.numpy as jnp
from jax import lax
from jax.experimental import pallas as pl
from jax.experimental.pallas import tpu as pltpu
from jax.experimental.pallas import tpu_sc as plsc

_N = 10000
_E = 160000
_ROWS = 10240
_NSUB = 16
_EPT = _E // _NSUB
_B = 32


def _lay1_body(x_ref, w_ref, bs_ref, bd_ref, ext_ref, adst_ref):
    t = jnp.dot(x_ref[...], w_ref[...], preferred_element_type=jnp.float32)
    asrc = jnp.dot(t, bs_ref[...], preferred_element_type=jnp.float32)
    ext_ref[...] = jnp.concatenate((t, asrc), axis=1)
    adst_ref[...] = jnp.dot(t, bd_ref[...], preferred_element_type=jnp.float32)


def _lay1(x, w, bsrc, bdst, bm=256):
    m, k = x.shape
    n = w.shape[1]
    return pl.pallas_call(
        _lay1_body,
        grid=(m // bm,),
        in_specs=[pl.BlockSpec((bm, k), lambda i: (i, 0)),
                  pl.BlockSpec((k, n), lambda i: (0, 0)),
                  pl.BlockSpec((n, 128), lambda i: (0, 0)),
                  pl.BlockSpec((n, 128), lambda i: (0, 0))],
        out_specs=[pl.BlockSpec((bm, n + 128), lambda i: (i, 0)),
                   pl.BlockSpec((bm, 128), lambda i: (i, 0))],
        out_shape=[jax.ShapeDtypeStruct((m, n + 128), jnp.float32),
                   jax.ShapeDtypeStruct((m, 128), jnp.float32)],
    )(x, w, bsrc, bdst)


def _lay2_body(acc_ref, s_ref, b_ref, w_ref, bs_ref, bd_ref,
               ext_ref, adst_ref):
    a = acc_ref[...]
    dr = jnp.dot(a[:, 1024:1040], s_ref[...],
                 preferred_element_type=jnp.float32) + 1e-16
    t = a[:, :1024] / dr + b_ref[...]
    t = jnp.where(t > 0, t, jnp.exp(jnp.minimum(t, 0.0)) - 1.0)
    h = jnp.dot(t, w_ref[...], preferred_element_type=jnp.float32)
    asrc = jnp.dot(h, bs_ref[...], preferred_element_type=jnp.float32)
    ext_ref[...] = jnp.concatenate((h, asrc), axis=1)
    adst_ref[...] = jnp.dot(h, bd_ref[...], preferred_element_type=jnp.float32)


def _lay2(acc, smat, bias, w, bsrc, bdst, bm=256):
    m = acc.shape[0]
    n = w.shape[1]
    return pl.pallas_call(
        _lay2_body,
        grid=(m // bm,),
        in_specs=[pl.BlockSpec((bm, 1152), lambda i: (i, 0)),
                  pl.BlockSpec((16, 1024), lambda i: (0, 0)),
                  pl.BlockSpec((1, 1024), lambda i: (0, 0)),
                  pl.BlockSpec((1024, n), lambda i: (0, 0)),
                  pl.BlockSpec((n, 128), lambda i: (0, 0)),
                  pl.BlockSpec((n, 128), lambda i: (0, 0))],
        out_specs=[pl.BlockSpec((bm, n + 128), lambda i: (i, 0)),
                   pl.BlockSpec((bm, 128), lambda i: (i, 0))],
        out_shape=[jax.ShapeDtypeStruct((m, n + 128), jnp.float32),
                   jax.ShapeDtypeStruct((m, 128), jnp.float32)],
    )(acc, smat, bias, w, bsrc, bdst)


def _fin_body(acc_ref, b_ref, o_ref):
    a = acc_ref[...]
    dr = a[:, 256:257] + 1e-16
    o_ref[...] = a[:, :256] / dr + b_ref[...]


def _fin(acc, bias, bm=256):
    m = acc.shape[0]
    return pl.pallas_call(
        _fin_body,
        grid=(m // bm,),
        in_specs=[pl.BlockSpec((bm, 384), lambda i: (i, 0)),
                  pl.BlockSpec((1, 256), lambda i: (0, 0))],
        out_specs=pl.BlockSpec((bm, 256), lambda i: (i, 0)),
        out_shape=jax.ShapeDtypeStruct((m, 256), jnp.float32),
    )(acc, bias)


def _make_sc_pass(F, H, CH, NCH):
    fph = F // H
    NS = F // 128 + 1
    mesh = plsc.VectorSubcoreMesh(core_axis_name="c", subcore_axis_name="s")
    wb = CH // _NSUB

    @functools.partial(
        pl.kernel, mesh=mesh,
        compiler_params=pltpu.CompilerParams(needs_layout_passes=False),
        out_type=tuple(jax.ShapeDtypeStruct((_ROWS, 128), jnp.float32)
                       for _ in range(NS)),
        scratch_types=[
            pltpu.VMEM((_EPT,), jnp.int32),
            pltpu.VMEM((_EPT,), jnp.int32),
            pltpu.VMEM((_EPT + 2 * _B,), jnp.int32),
            pltpu.VMEM((_B,), jnp.int32),
            pltpu.VMEM((_B,), jnp.int32),
            pltpu.VMEM((_B,), jnp.int32),
            pltpu.VMEM((_B, 128), jnp.float32),
        ] + [pltpu.VMEM((_B, 128), jnp.float32) for _ in range(NS)]
          + [pltpu.VMEM_SHARED((CH, 128), jnp.float32) for _ in range(NS)]
          + [
            pltpu.SemaphoreType.DMA,
            pltpu.SemaphoreType.DMA,
            pltpu.SemaphoreType.DMA,
        ])
    def sc_pass(src_hbm, dst_hbm, adst_hbm, z_hbm, *rest):
        ext_s = rest[:NS]
        acc_out = rest[NS:2 * NS]
        (src_vm, dst_vm, sel_vm, srci_vm, dsti_vm, dloc_vm,
         adst_vm) = rest[2 * NS:2 * NS + 7]
        rows_s = rest[2 * NS + 7:3 * NS + 7]
        acc_s = rest[3 * NS + 7:4 * NS + 7]
        sem0, sem1, sem2 = rest[4 * NS + 7:]
        cid = lax.axis_index("c")
        sid = lax.axis_index("s")
        ebase = sid * _EPT
        pltpu.sync_copy(src_hbm.at[pl.ds(ebase, _EPT)], src_vm)
        pltpu.sync_copy(dst_hbm.at[pl.ds(ebase, _EPT)], dst_vm)
        iota16 = lax.iota(jnp.int32, 16)
        r0 = sid * wb

        def chunk_body(k, _):
            c0 = (cid * NCH + k) * CH
            for c in range(NS):
                pltpu.sync_copy(z_hbm.at[pl.ds(r0, wb)],
                                acc_s[c].at[pl.ds(r0, wb)])
            plsc.subcore_barrier()

            def fbody(i, cnt):
                d = dst_vm[pl.ds(i * 16, 16)]
                m = (d >= c0) & (d < c0 + CH)
                mi = m.astype(jnp.int32)
                pos = cnt + plsc.cumsum(mi) - 1
                plsc.store_scatter(sel_vm, [pos], iota16 + i * 16, mask=m)
                return cnt + jnp.sum(mi)
            cnt = lax.fori_loop(0, _EPT // 16, fbody, jnp.int32(0))
            for t in range(_B // 16):
                plsc.store_scatter(sel_vm, [cnt + iota16 + t * 16],
                                   jnp.zeros((16,), jnp.int32))
            nb = (cnt + _B - 1) // _B

            def batch_body(j, _):
                base = j * _B
                for g in range(_B // 16):
                    lidx = sel_vm[pl.ds(base + g * 16, 16)]
                    s = plsc.load_gather(src_vm, [lidx])
                    d = plsc.load_gather(dst_vm, [lidx])
                    dl = jnp.minimum(jnp.maximum(d - c0, 0), CH - 1)
                    srci_vm[pl.ds(g * 16, 16)] = s
                    dsti_vm[pl.ds(g * 16, 16)] = d
                    dloc_vm[pl.ds(g * 16, 16)] = dl
                ca = pltpu.async_copy(adst_hbm.at[dsti_vm], adst_vm, sem0)
                gs = [pltpu.async_copy(ext_s[c].at[srci_vm], rows_s[c], sem1)
                      for c in range(NS)]
                ca.wait()
                for g in gs:
                    g.wait()
                wbuf = rows_s[NS - 1]
                for g in range(_B // 16):
                    rowi = iota16 + g * 16
                    valid = (base + rowi) < cnt
                    for hh in range(H):
                        a_s = plsc.load_gather(
                            wbuf, [rowi, jnp.full((16,), hh, jnp.int32)])
                        a_d = plsc.load_gather(
                            adst_vm, [rowi, jnp.full((16,), hh, jnp.int32)])
                        sc = a_s + a_d
                        sc = jnp.where(sc >= 0, sc, 0.2 * sc)
                        wv = jnp.where(valid, jnp.exp(sc), 0.0)
                        plsc.store_scatter(
                            wbuf, [rowi, jnp.full((16,), hh, jnp.int32)], wv)

                @plsc.parallel_loop(0, _B, 1, unroll=4)
                def _scale(b):
                    wrow = wbuf[b, pl.ds(0, 16)]
                    for c in range(NS - 1):
                        wsv = jnp.full((16,), wrow[(c * 128) // fph],
                                       jnp.float32)
                        for jv in range(8):
                            rows_s[c][b, pl.ds(jv * 16, 16)] = (
                                rows_s[c][b, pl.ds(jv * 16, 16)] * wsv)
                ss = [pltpu.async_copy(rows_s[c], acc_s[c].at[dloc_vm], sem2,
                                       add=True)
                      for c in range(NS)]
                for g in ss:
                    g.wait()
                return 0
            lax.fori_loop(0, nb, batch_body, 0)
            plsc.subcore_barrier()

            for c in range(NS):
                pltpu.sync_copy(acc_s[c].at[pl.ds(r0, wb)],
                                acc_out[c].at[pl.ds(c0 + r0, wb)])
            return 0
        lax.fori_loop(0, NCH, chunk_body, 0)

    return sc_pass


_sc_pass_1 = _make_sc_pass(1024, 8, 128, 40)
_sc_pass_2 = _make_sc_pass(256, 1, 512, 10)


def _logit_mats(a_src, a_dst, f_tot, heads):
    fph = f_tot // heads
    bs = jnp.zeros((f_tot, 128), jnp.float32)
    bd = jnp.zeros((f_tot, 128), jnp.float32)
    for h in range(heads):
        bs = bs.at[h * fph:(h + 1) * fph, h].set(a_src[h])
        bd = bd.at[h * fph:(h + 1) * fph, h].set(a_dst[h])
    return bs, bd


def _spread_mat(f_tot, heads):
    fph = f_tot // heads
    s = jnp.zeros((16, f_tot), jnp.float32)
    for h in range(heads):
        s = s.at[h, h * fph:(h + 1) * fph].set(1.0)
    return s


def kernel(x, adj, W1, a_src1, a_dst1, b1, W2, a_src2, a_dst2, b2):
    src = adj[0]
    dst = adj[1]
    bs1, bd1 = _logit_mats(a_src1, a_dst1, 1024, 8)
    bs2, bd2 = _logit_mats(a_src2, a_dst2, 256, 1)
    s1m = _spread_mat(1024, 8)
    z1 = jnp.zeros((128, 128), jnp.float32)
    z2 = jnp.zeros((512, 128), jnp.float32)

    xp = jnp.pad(x, ((0, 10240 - _N), (0, 0)))
    ext1, ad1 = _lay1(xp, W1, bs1, bd1)
    e1s = [ext1[:, c * 128:(c + 1) * 128] for c in range(9)]
    acc1 = jnp.concatenate(
        _sc_pass_1(src, dst, ad1, z1, *e1s), axis=1)
    ext2, ad2 = _lay2(acc1, s1m, b1.reshape(1, -1), W2, bs2, bd2)
    e2s = [ext2[:, c * 128:(c + 1) * 128] for c in range(3)]
    acc2 = jnp.concatenate(
        _sc_pass_2(src, dst, ad2, z2, *e2s), axis=1)
    out = _fin(acc2, b2.reshape(1, -1))
    return out[:_N]
